# baseline (speedup 1.0000x reference)
import functools

import jax
import jax.numpy as jnp
from jax.experimental import pallas as pl
from jax.experimental.pallas import tpu as pltpu
from jax.sharding import PartitionSpec as P
from jax.experimental.shard_map import shard_map


def _gather_ce_kernel(idx_ref, tgt_ref, table_ref, logits_ref, tokloss_ref,
                      rows_ref, *, tile_n, v):
    for mi in range(tile_n):
        rows_ref[mi, 0] = table_ref[idx_ref[0, 0, mi], 0]

    logits_ref[...] = rows_ref[...].reshape(tile_n, v)

    vals = logits_ref[...]
    col = jax.lax.broadcasted_iota(jnp.int32, (tile_n, v), 1)
    m = jnp.max(vals, axis=-1, keepdims=True)
    lse = m + jnp.log(jnp.sum(jnp.exp(vals - m), axis=-1, keepdims=True))
    tgt_logit = jnp.sum(jnp.where(col == tgt_ref[...], vals, 0.0),
                        axis=-1, keepdims=True)
    tokloss_ref[...] = lse - tgt_logit


def _forward_local(idx_part, tgt_part, table3, *, tile_n, v):
    n_loc = idx_part.size
    num_tiles = n_loc // tile_n

    idx_rows = idx_part.reshape(num_tiles, 1, tile_n).astype(jnp.int32)
    tgt_col = tgt_part.reshape(n_loc, 1).astype(jnp.int32)

    body = functools.partial(_gather_ce_kernel, tile_n=tile_n, v=v)
    return pl.pallas_call(
        body,
        grid=(num_tiles,),
        out_shape=(
            jax.ShapeDtypeStruct((n_loc, v), jnp.float32),
            jax.ShapeDtypeStruct((n_loc, 1), jnp.float32),
        ),
        in_specs=[
            pl.BlockSpec((1, 1, tile_n), lambda i: (i, 0, 0),
                         memory_space=pltpu.SMEM),
            pl.BlockSpec((tile_n, 1), lambda i: (i, 0)),
            pl.BlockSpec((v, 1, v), lambda i: (0, 0, 0)),
        ],
        out_specs=(
            pl.BlockSpec((tile_n, v), lambda i: (i, 0)),
            pl.BlockSpec((tile_n, 1), lambda i: (i, 0)),
        ),
        scratch_shapes=[pltpu.VMEM((tile_n, 1, v), jnp.float32)],
        compiler_params=pltpu.CompilerParams(
            dimension_semantics=("parallel",)),
    )(idx_rows, tgt_col, table3)


def kernel(idx, embedding_table, targets):
    B, T = idx.shape
    V = embedding_table.shape[0]
    N = B * T

    tile_n = 256
    assert V % 128 == 0

    ndev = 2 if jax.local_device_count() >= 2 else 1
    assert (N // ndev) % tile_n == 0 and B % ndev == 0

    table3 = embedding_table.reshape(V, 1, V)
    mesh = jax.make_mesh((ndev,), ("x",))

    def shard_fn(idx_s, tgt_s, table3_s):
        logits_l, tokloss_l = _forward_local(
            idx_s, tgt_s, table3_s, tile_n=tile_n, v=V)
        loss_sum = jax.lax.psum(jnp.sum(tokloss_l), "x")
        return logits_l, loss_sum

    sharded = shard_map(
        shard_fn, mesh=mesh,
        in_specs=(P("x"), P("x"), P(None, None, None)),
        out_specs=(P("x"), P()),
        check_rep=False,
    )
    idx = jax.reshard(idx, jax.sharding.NamedSharding(mesh, P("x")))
    targets = jax.reshard(targets, jax.sharding.NamedSharding(mesh, P("x")))
    table3 = jax.reshard(
        table3, jax.sharding.NamedSharding(mesh, P(None, None, None)))
    logits, loss_sum = sharded(idx, targets, table3)
    return logits, loss_sum / N
